# baseline (speedup 1.0000x reference)
import functools

import jax
import jax.numpy as jnp
from jax import lax
from jax.experimental import pallas as pl
from jax.experimental.pallas import tpu as pltpu
from jax.experimental.pallas import tpu_sc as plsc

D_MODEL = 64
NC = 2
NS = 16
NW = NC * NS
BLK = 128
G = 2


def _body(l, blocks, xt_hbm, table_hbm, out_hbm, idx_v, rows0, rows1,
          tb0, tb1, gsem0, gsem1, ssem0, ssem1):
    wid = lax.axis_index("s") * NC + lax.axis_index("c")
    rows = (rows0, rows1)
    tb = (tb0, tb1)
    gsem = (gsem0, gsem1)
    ssem = (ssem0, ssem1)
    iota = lax.iota(jnp.int32, 16)

    pltpu.sync_copy(xt_hbm.at[:, pl.ds(wid * BLK, BLK)], idx_v)

    def gather_copies(g, p):
        return [
            pltpu.make_async_copy(
                table_hbm.at[idx_v.at[g * G + j]],
                rows[p].at[pl.ds(j * BLK, BLK)],
                gsem[p],
            )
            for j in range(G)
        ]

    def fire_gathers(g, p):
        for c in gather_copies(g, p):
            c.start()

    def wait_gathers(g, p):
        for c in gather_copies(g, p):
            c.wait()

    def store_copies(g, p):
        return [
            pltpu.make_async_copy(
                tb[p].at[j, dh], out_hbm.at[g * G + j, dh, wid], ssem[p]
            )
            for j in range(G)
            for dh in range(8)
        ]

    def wait_stores(g, p):
        for c in store_copies(g, p):
            c.wait()

    def transpose_and_store(g, p):
        @pl.loop(0, G * 8)
        def _(k):
            j = k // 8
            dh = k % 8
            base = j * BLK
            for dl in range(8):
                d = dh * 8 + dl
                for q in range(8):
                    ridx = iota + (base + q * 16)
                    cidx = jnp.full((16,), d, jnp.int32)
                    vec = plsc.load_gather(rows[p], [ridx, cidx])
                    tb[p][j, dh, dl, pl.ds(q * 16, 16)] = vec
            pltpu.make_async_copy(
                tb[p].at[j, dh], out_hbm.at[g * G + j, dh, wid], ssem[p]
            ).start()

    fire_gathers(0, 0)
    fire_gathers(1, 1)
    wait_gathers(0, 0)
    transpose_and_store(0, 0)
    fire_gathers(2, 0)
    wait_gathers(1, 1)
    transpose_and_store(1, 1)

    @pl.loop(2, blocks - 2, step=2)
    def _(g0):
        for b in range(2):
            g = g0 + b
            p = b % 2
            fire_gathers(g + 1, (b + 1) % 2)
            wait_gathers(g, p)
            wait_stores(g - 2, p)
            transpose_and_store(g, p)

    last = blocks - 1
    fire_gathers(last, 1)
    wait_gathers(last - 1, 0)
    wait_stores(last - 3, 0)
    transpose_and_store(last - 1, 0)
    wait_gathers(last, 1)
    wait_stores(last - 2, 1)
    transpose_and_store(last, 1)
    wait_stores(last - 1, 0)
    wait_stores(last, 1)


@jax.jit
def kernel(x, table):
    b, l = x.shape
    assert b == NW * BLK and l % (2 * G) == 0
    blocks = l // G
    xt = (x.astype(jnp.int32) * 2).T
    v = table.shape[0]
    tv = jnp.pad(table, ((0, 0), (0, 128 - D_MODEL))).reshape(2 * v, D_MODEL)

    mesh = plsc.VectorSubcoreMesh(
        core_axis_name="c", subcore_axis_name="s", num_cores=NC, num_subcores=NS
    )
    out5 = pl.kernel(
        functools.partial(_body, l, blocks),
        out_type=jax.ShapeDtypeStruct((l, 8, NW, 8, BLK), jnp.float32),
        mesh=mesh,
        scratch_types=[
            pltpu.VMEM((l, BLK), jnp.int32),
            pltpu.VMEM((G * BLK, D_MODEL), jnp.float32),
            pltpu.VMEM((G * BLK, D_MODEL), jnp.float32),
            pltpu.VMEM((G, 8, 8, BLK), jnp.float32),
            pltpu.VMEM((G, 8, 8, BLK), jnp.float32),
            pltpu.SemaphoreType.DMA,
            pltpu.SemaphoreType.DMA,
            pltpu.SemaphoreType.DMA,
            pltpu.SemaphoreType.DMA,
        ],
        compiler_params=pltpu.CompilerParams(
            use_tc_tiling_on_sc=False, needs_layout_passes=False
        ),
    )(xt, tv)
    return out5.transpose(2, 4, 0, 1, 3).reshape(b, l, D_MODEL)


# speedup vs baseline: 2.0522x; 2.0522x over previous
import functools

import jax
import jax.numpy as jnp
from jax import lax
from jax.experimental import pallas as pl
from jax.experimental.pallas import tpu as pltpu
from jax.experimental.pallas import tpu_sc as plsc

D_MODEL = 64
NC = 2
NS = 16
NW = NC * NS

NB = 2
SPLITS = (0, 128)


def _body(l, bpw, blocks, x_hbm, table_hbm, out_hbm, idx_v, rows0, rows1,
          gsem0, gsem1, ssem0, ssem1):
    wid = lax.axis_index("s") * NC + lax.axis_index("c")
    b0 = wid * bpw
    rows = (rows0, rows1)
    gsem = (gsem0, gsem1)
    ssem = (ssem0, ssem1)
    widths = [SPLITS[i + 1] - SPLITS[i] if i + 1 < len(SPLITS) else l - SPLITS[i]
              for i in range(len(SPLITS))]

    pltpu.sync_copy(x_hbm.at[pl.ds(b0, bpw)], idx_v)

    def gather_copies(g, p):
        return [
            pltpu.make_async_copy(
                table_hbm.at[idx_v.at[g * NB + i, pl.ds(s, w)]],
                rows[p].at[pl.ds(i * l + s, w)],
                gsem[p],
            )
            for i in range(NB)
            for s, w in zip(SPLITS, widths)
        ]

    def fire_gathers(g, p):
        for c in gather_copies(g, p):
            c.start()

    def wait_gathers(g, p):
        for c in gather_copies(g, p):
            c.wait()

    def store_copy(g, p):
        return pltpu.make_async_copy(
            rows[p],
            out_hbm.at[pl.ds((b0 + g * NB) * l, NB * l), pl.ds(0, D_MODEL)],
            ssem[p],
        )

    fire_gathers(0, 0)
    fire_gathers(1, 1)
    wait_gathers(0, 0)
    store_copy(0, 0).start()

    @pl.loop(1, blocks - 1, step=2)
    def _(g0):
        for b in range(2):
            g = g0 + b
            p = (1 + b) % 2
            np_ = b % 2
            store_copy(g - 1, np_).wait()
            fire_gathers(g + 1, np_)
            wait_gathers(g, p)
            store_copy(g, p).start()

    last = blocks - 1
    wait_gathers(last, last % 2)
    store_copy(last, last % 2).start()
    store_copy(last - 1, (last - 1) % 2).wait()
    store_copy(last, last % 2).wait()


@jax.jit
def kernel(x, table):
    b, l = x.shape
    assert b % (NW * NB) == 0
    bpw = b // NW
    blocks = bpw // NB
    assert blocks % 2 == 0
    xi = x.astype(jnp.int32) * 2
    v = table.shape[0]
    tpad = jnp.pad(table, ((0, 0), (0, 128 - D_MODEL)))
    tv = tpad.reshape(2 * v, D_MODEL)

    mesh = plsc.VectorSubcoreMesh(
        core_axis_name="c", subcore_axis_name="s", num_cores=NC, num_subcores=NS
    )
    out = pl.kernel(
        functools.partial(_body, l, bpw, blocks),
        out_type=jax.ShapeDtypeStruct((b * l, 128), jnp.float32),
        mesh=mesh,
        scratch_types=[
            pltpu.VMEM((b // NW, l), jnp.int32),
            pltpu.VMEM((NB * l, D_MODEL), jnp.float32),
            pltpu.VMEM((NB * l, D_MODEL), jnp.float32),
            pltpu.SemaphoreType.DMA,
            pltpu.SemaphoreType.DMA,
            pltpu.SemaphoreType.DMA,
            pltpu.SemaphoreType.DMA,
        ],
        compiler_params=pltpu.CompilerParams(use_tc_tiling_on_sc=False),
    )(xi, tv)
    return out[:, :D_MODEL].reshape(b, l, D_MODEL)
